# baseline (speedup 1.0000x reference)
import functools

import jax
import jax.numpy as jnp
from jax import lax
from jax.experimental import pallas as pl
from jax.experimental.pallas import tpu as pltpu
from jax.experimental.pallas import tpu_sc as plsc

_IDX_CHUNK = 128


def _sc_gather(idx3d, emb_table):
    num_workers, chunks_per_w, chunk = idx3d.shape
    _, emb = emb_table.shape

    mesh = plsc.VectorSubcoreMesh(core_axis_name="c", subcore_axis_name="s")

    @functools.partial(
        pl.kernel,
        out_type=jax.ShapeDtypeStruct((num_workers * chunks_per_w, chunk, emb), jnp.float32),
        mesh=mesh,
        scratch_types=[
            pltpu.VMEM((chunks_per_w, chunk), jnp.int32),
            pltpu.VMEM((chunks_per_w, chunk, emb), jnp.float32),
            pltpu.SemaphoreType.DMA,
        ],
        compiler_params=pltpu.CompilerParams(use_tc_tiling_on_sc=False),
    )
    def gather_kernel(idx_hbm, table_hbm, out_hbm, idx_v, rows_v, sem):
        num_cores = jax.lax.axis_size("c")
        wid = lax.axis_index("s") * num_cores + lax.axis_index("c")
        pltpu.sync_copy(idx_hbm.at[wid], idx_v)
        copies = [
            pltpu.async_copy(table_hbm.at[idx_v.at[j]], rows_v.at[j], sem)
            for j in range(chunks_per_w)
        ]
        for c in copies:
            c.wait()
        pltpu.sync_copy(rows_v, out_hbm.at[pl.ds(wid * chunks_per_w, chunks_per_w)])

    return gather_kernel(idx3d, emb_table)


def _projection(flat, dense_w, dense_b, block_n, n_out_bufs=3):
    m, k = flat.shape
    n = dense_w.shape[1]
    nsteps = n // block_n
    n_main = nsteps * block_n
    bias_rows = lax.slice(dense_b, (0,), (n_main,)).reshape(nsteps, 1, block_n)

    def mm_kernel(flat_ref, w_hbm, b_ref, out_hbm, w_buf, o_buf, w_sem, o_sem):
        i = pl.program_id(0)

        def w_copy(step, slot):
            return pltpu.make_async_copy(
                w_hbm.at[:, pl.ds(step * block_n, block_n)],
                w_buf.at[slot],
                w_sem.at[slot],
            )

        def o_copy(step, slot):
            return pltpu.make_async_copy(
                o_buf.at[slot],
                out_hbm.at[:, pl.ds(step * block_n, block_n)],
                o_sem.at[slot],
            )

        @pl.when(i == 0)
        def _():
            w_copy(0, 0).start()

        @pl.when(i + 1 < nsteps)
        def _():
            w_copy(i + 1, (i + 1) % 2).start()

        slot_w = i % 2
        slot_o = i % n_out_bufs

        @pl.when(i >= n_out_bufs)
        def _():
            o_copy(i - n_out_bufs, slot_o).wait()

        w_copy(i, slot_w).wait()
        o_buf[slot_o] = (
            jnp.dot(flat_ref[...], w_buf[slot_w], preferred_element_type=jnp.float32)
            + b_ref[0]
        )
        o_copy(i, slot_o).start()

        @pl.when(i == nsteps - 1)
        def _():
            for d in range(n_out_bufs):
                step = nsteps - n_out_bufs + d
                o_copy(step, step % n_out_bufs).wait()

    return pl.pallas_call(
        mm_kernel,
        grid=(nsteps,),
        in_specs=[
            pl.BlockSpec((m, k), lambda i: (0, 0)),
            pl.BlockSpec(memory_space=pl.ANY),
            pl.BlockSpec((1, 1, block_n), lambda i: (i, 0, 0)),
        ],
        out_specs=pl.BlockSpec(memory_space=pl.ANY),
        out_shape=jax.ShapeDtypeStruct((m, n), jnp.float32),
        scratch_shapes=[
            pltpu.VMEM((2, k, block_n), jnp.float32),
            pltpu.VMEM((n_out_bufs, m, block_n), jnp.float32),
            pltpu.SemaphoreType.DMA((2,)),
            pltpu.SemaphoreType.DMA((n_out_bufs,)),
        ],
        compiler_params=pltpu.CompilerParams(
            dimension_semantics=("arbitrary",),
        ),
    )(flat, dense_w, bias_rows)


def _projection_tail(flat, dense_w, dense_b, out, block_n):
    m, k = flat.shape
    n = dense_w.shape[1]
    nsteps = n // block_n
    n_tail = n - nsteps * block_n
    bias_tail = jnp.pad(
        lax.slice(dense_b, (nsteps * block_n,), (n,)), (0, block_n - n_tail)
    ).reshape(1, 1, block_n)

    def tail_kernel(flat_ref, w_ref, b_ref, prev_ref, out_ref):
        del prev_ref
        out_ref[...] = (
            jnp.dot(flat_ref[...], w_ref[...], preferred_element_type=jnp.float32)
            + b_ref[0]
        )

    return pl.pallas_call(
        tail_kernel,
        grid=(1,),
        in_specs=[
            pl.BlockSpec((m, k), lambda i: (0, 0)),
            pl.BlockSpec((k, block_n), lambda i: (0, nsteps)),
            pl.BlockSpec((1, 1, block_n), lambda i: (0, 0, 0)),
            pl.BlockSpec(memory_space=pl.ANY),
        ],
        out_specs=pl.BlockSpec((m, block_n), lambda i: (0, nsteps)),
        out_shape=jax.ShapeDtypeStruct((m, n), jnp.float32),
        input_output_aliases={3: 0},
    )(flat, dense_w, bias_tail, out)


def kernel(context, emb_table, dense_w, dense_b):
    batch, ctx_len = context.shape
    vocab, emb = emb_table.shape
    total = batch * ctx_len
    info = plsc.get_sparse_core_info()
    num_workers = info.num_cores * info.num_subcores
    idx3d = context.astype(jnp.int32).reshape(
        num_workers, total // (num_workers * _IDX_CHUNK), _IDX_CHUNK
    )
    rows = _sc_gather(idx3d, emb_table)
    flat = rows.reshape(batch, ctx_len * emb)
    logits = _projection(flat, dense_w, dense_b, block_n=2048)
    logits = _projection_tail(flat, dense_w, dense_b, logits, block_n=2048)
    return logits


# speedup vs baseline: 1.0011x; 1.0011x over previous
import functools

import jax
import jax.numpy as jnp
from jax import lax
from jax.experimental import pallas as pl
from jax.experimental.pallas import tpu as pltpu
from jax.experimental.pallas import tpu_sc as plsc

_IDX_CHUNK = 128


def _sc_gather(idx3d, emb_table):
    num_workers, chunks_per_w, chunk = idx3d.shape
    _, emb = emb_table.shape

    mesh = plsc.VectorSubcoreMesh(core_axis_name="c", subcore_axis_name="s")

    @functools.partial(
        pl.kernel,
        out_type=jax.ShapeDtypeStruct((num_workers * chunks_per_w, chunk, emb), jnp.float32),
        mesh=mesh,
        scratch_types=[
            pltpu.VMEM((chunks_per_w, chunk), jnp.int32),
            pltpu.VMEM((chunks_per_w, chunk, emb), jnp.float32),
            pltpu.SemaphoreType.DMA,
        ],
        compiler_params=pltpu.CompilerParams(use_tc_tiling_on_sc=False),
    )
    def gather_kernel(idx_hbm, table_hbm, out_hbm, idx_v, rows_v, sem):
        num_cores = jax.lax.axis_size("c")
        wid = lax.axis_index("s") * num_cores + lax.axis_index("c")
        pltpu.sync_copy(idx_hbm.at[wid], idx_v)
        copies = [
            pltpu.async_copy(table_hbm.at[idx_v.at[j]], rows_v.at[j], sem)
            for j in range(chunks_per_w)
        ]
        for c in copies:
            c.wait()
        pltpu.sync_copy(rows_v, out_hbm.at[pl.ds(wid * chunks_per_w, chunks_per_w)])

    return gather_kernel(idx3d, emb_table)


def _projection(flat, dense_w, dense_b, block_n, n_out_bufs=4):
    m, k = flat.shape
    n = dense_w.shape[1]
    nsteps = n // block_n
    n_main = nsteps * block_n
    bias_rows = lax.slice(dense_b, (0,), (n_main,)).reshape(nsteps, 1, block_n)

    def mm_kernel(flat_ref, w_hbm, b_ref, out_hbm, w_buf, o_buf, w_sem, o_sem):
        i = pl.program_id(0)

        def w_copy(step, slot):
            return pltpu.make_async_copy(
                w_hbm.at[:, pl.ds(step * block_n, block_n)],
                w_buf.at[slot],
                w_sem.at[slot],
            )

        def o_copy(step, slot):
            return pltpu.make_async_copy(
                o_buf.at[slot],
                out_hbm.at[:, pl.ds(step * block_n, block_n)],
                o_sem.at[slot],
            )

        @pl.when(i == 0)
        def _():
            w_copy(0, 0).start(priority=0)

        for s in range(2):
            @pl.when((i + 1 < nsteps) & ((i + 1) % 2 == s))
            def _(s=s):
                w_copy(i + 1, s).start(priority=s)

        slot_w = i % 2
        slot_o = i % n_out_bufs

        @pl.when(i >= n_out_bufs)
        def _():
            o_copy(i - n_out_bufs, slot_o).wait()

        w_copy(i, slot_w).wait()
        o_buf[slot_o] = (
            jnp.dot(flat_ref[...], w_buf[slot_w], preferred_element_type=jnp.float32)
            + b_ref[0]
        )
        for s in range(n_out_bufs):
            @pl.when(slot_o == s)
            def _(s=s):
                o_copy(i, s).start(priority=s % 2)

        @pl.when(i == nsteps - 1)
        def _():
            for d in range(n_out_bufs):
                step = nsteps - n_out_bufs + d
                o_copy(step, step % n_out_bufs).wait()

    return pl.pallas_call(
        mm_kernel,
        grid=(nsteps,),
        in_specs=[
            pl.BlockSpec((m, k), lambda i: (0, 0)),
            pl.BlockSpec(memory_space=pl.ANY),
            pl.BlockSpec((1, 1, block_n), lambda i: (i, 0, 0)),
        ],
        out_specs=pl.BlockSpec(memory_space=pl.ANY),
        out_shape=jax.ShapeDtypeStruct((m, n), jnp.float32),
        scratch_shapes=[
            pltpu.VMEM((2, k, block_n), jnp.float32),
            pltpu.VMEM((n_out_bufs, m, block_n), jnp.float32),
            pltpu.SemaphoreType.DMA((2,)),
            pltpu.SemaphoreType.DMA((n_out_bufs,)),
        ],
        compiler_params=pltpu.CompilerParams(
            dimension_semantics=("arbitrary",),
        ),
    )(flat, dense_w, bias_rows)


def _projection_tail(flat, dense_w, dense_b, out, block_n):
    m, k = flat.shape
    n = dense_w.shape[1]
    nsteps = n // block_n
    n_tail = n - nsteps * block_n
    bias_tail = jnp.pad(
        lax.slice(dense_b, (nsteps * block_n,), (n,)), (0, block_n - n_tail)
    ).reshape(1, 1, block_n)

    def tail_kernel(flat_ref, w_ref, b_ref, prev_ref, out_ref):
        del prev_ref
        out_ref[...] = (
            jnp.dot(flat_ref[...], w_ref[...], preferred_element_type=jnp.float32)
            + b_ref[0]
        )

    return pl.pallas_call(
        tail_kernel,
        grid=(1,),
        in_specs=[
            pl.BlockSpec((m, k), lambda i: (0, 0)),
            pl.BlockSpec((k, block_n), lambda i: (0, nsteps)),
            pl.BlockSpec((1, 1, block_n), lambda i: (0, 0, 0)),
            pl.BlockSpec(memory_space=pl.ANY),
        ],
        out_specs=pl.BlockSpec((m, block_n), lambda i: (0, nsteps)),
        out_shape=jax.ShapeDtypeStruct((m, n), jnp.float32),
        input_output_aliases={3: 0},
    )(flat, dense_w, bias_tail, out)


def kernel(context, emb_table, dense_w, dense_b):
    batch, ctx_len = context.shape
    vocab, emb = emb_table.shape
    total = batch * ctx_len
    info = plsc.get_sparse_core_info()
    num_workers = info.num_cores * info.num_subcores
    idx3d = context.astype(jnp.int32).reshape(
        num_workers, total // (num_workers * _IDX_CHUNK), _IDX_CHUNK
    )
    rows = _sc_gather(idx3d, emb_table)
    flat = rows.reshape(batch, ctx_len * emb)
    logits = _projection(flat, dense_w, dense_b, block_n=2048)
    logits = _projection_tail(flat, dense_w, dense_b, logits, block_n=2048)
    return logits


# speedup vs baseline: 1.1181x; 1.1169x over previous
import functools

import jax
import jax.numpy as jnp
from jax import lax
from jax.experimental import pallas as pl
from jax.experimental.pallas import tpu as pltpu
from jax.experimental.pallas import tpu_sc as plsc

_IDX_CHUNK = 128


def _sc_gather(idx3d, emb_table):
    num_workers, chunks_per_w, chunk = idx3d.shape
    _, emb = emb_table.shape

    mesh = plsc.VectorSubcoreMesh(core_axis_name="c", subcore_axis_name="s")

    @functools.partial(
        pl.kernel,
        out_type=jax.ShapeDtypeStruct((num_workers * chunks_per_w, chunk, emb), jnp.float32),
        mesh=mesh,
        scratch_types=[
            pltpu.VMEM((chunks_per_w, chunk), jnp.int32),
            pltpu.VMEM((chunks_per_w, chunk, emb), jnp.float32),
            pltpu.SemaphoreType.DMA,
        ],
        compiler_params=pltpu.CompilerParams(use_tc_tiling_on_sc=False),
    )
    def gather_kernel(idx_hbm, table_hbm, out_hbm, idx_v, rows_v, sem):
        num_cores = jax.lax.axis_size("c")
        wid = lax.axis_index("s") * num_cores + lax.axis_index("c")
        pltpu.sync_copy(idx_hbm.at[wid], idx_v)
        copies = [
            pltpu.async_copy(table_hbm.at[idx_v.at[j]], rows_v.at[j], sem)
            for j in range(chunks_per_w)
        ]
        for c in copies:
            c.wait()
        pltpu.sync_copy(rows_v, out_hbm.at[pl.ds(wid * chunks_per_w, chunks_per_w)])

    return gather_kernel(idx3d, emb_table)


def _projection(flat, dense_w, dense_b, block_n, n_out_bufs=4):
    m, k = flat.shape
    n = dense_w.shape[1]
    nsteps = n // block_n
    n_main = nsteps * block_n
    bias_rows = lax.slice(dense_b, (0,), (n_main,)).reshape(nsteps, 1, block_n)

    def mm_kernel(flat_ref, w_hbm, b_ref, out_hbm, w_buf, o_buf, w_sem, o_sem):
        i = pl.program_id(0)

        def w_copy(step, slot):
            return pltpu.make_async_copy(
                w_hbm.at[:, pl.ds(step * block_n, block_n)],
                w_buf.at[slot],
                w_sem.at[slot],
            )

        def o_copy(step, slot):
            return pltpu.make_async_copy(
                o_buf.at[slot],
                out_hbm.at[:, pl.ds(step * block_n, block_n)],
                o_sem.at[slot],
            )

        @pl.when(i == 0)
        def _():
            w_copy(0, 0).start(priority=0)

        for s in range(2):
            @pl.when((i + 1 < nsteps) & ((i + 1) % 2 == s))
            def _(s=s):
                w_copy(i + 1, s).start(priority=s)

        slot_w = i % 2
        slot_o = i % n_out_bufs

        @pl.when(i >= n_out_bufs)
        def _():
            o_copy(i - n_out_bufs, slot_o).wait()

        w_copy(i, slot_w).wait()
        o_buf[slot_o] = (
            jnp.dot(flat_ref[...], w_buf[slot_w], preferred_element_type=jnp.float32)
            + b_ref[0]
        )
        for s in range(n_out_bufs):
            @pl.when(slot_o == s)
            def _(s=s):
                o_copy(i, s).start(priority=s % 2)

        @pl.when(i == nsteps - 1)
        def _():
            for d in range(n_out_bufs):
                step = nsteps - n_out_bufs + d
                o_copy(step, step % n_out_bufs).wait()

    return pl.pallas_call(
        mm_kernel,
        grid=(nsteps,),
        in_specs=[
            pl.BlockSpec((m, k), lambda i: (0, 0)),
            pl.BlockSpec(memory_space=pl.ANY),
            pl.BlockSpec((1, 1, block_n), lambda i: (i, 0, 0)),
        ],
        out_specs=pl.BlockSpec(memory_space=pl.ANY),
        out_shape=jax.ShapeDtypeStruct((m, n), jnp.float32),
        scratch_shapes=[
            pltpu.VMEM((2, k, block_n), jnp.float32),
            pltpu.VMEM((n_out_bufs, m, block_n), jnp.float32),
            pltpu.SemaphoreType.DMA((2,)),
            pltpu.SemaphoreType.DMA((n_out_bufs,)),
        ],
        compiler_params=pltpu.CompilerParams(
            dimension_semantics=("arbitrary",),
        ),
    )(flat, dense_w, bias_rows)


def _projection_tail(flat, dense_w, dense_b, block_n):
    m, k = flat.shape
    n = dense_w.shape[1]
    nsteps = n // block_n
    n_tail = n - nsteps * block_n
    w_tail = lax.slice(dense_w, (0, nsteps * block_n), (k, n))
    bias_tail = lax.slice(dense_b, (nsteps * block_n,), (n,)).reshape(1, 1, n_tail)

    def tail_kernel(flat_ref, w_ref, b_ref, out_ref):
        out_ref[...] = (
            jnp.dot(flat_ref[...], w_ref[...], preferred_element_type=jnp.float32)
            + b_ref[0]
        )

    return pl.pallas_call(
        tail_kernel,
        out_shape=jax.ShapeDtypeStruct((m, n_tail), jnp.float32),
    )(flat, w_tail, bias_tail)


def kernel(context, emb_table, dense_w, dense_b):
    batch, ctx_len = context.shape
    vocab, emb = emb_table.shape
    total = batch * ctx_len
    info = plsc.get_sparse_core_info()
    num_workers = info.num_cores * info.num_subcores
    idx3d = context.astype(jnp.int32).reshape(
        num_workers, total // (num_workers * _IDX_CHUNK), _IDX_CHUNK
    )
    rows = _sc_gather(idx3d, emb_table)
    flat = rows.reshape(batch, ctx_len * emb)
    logits = _projection(flat, dense_w, dense_b, block_n=2048)
    tail = _projection_tail(flat, dense_w, dense_b, block_n=2048)
    n_main = (vocab // 2048) * 2048
    logits = lax.dynamic_update_slice(logits, tail, (0, n_main))
    return logits
